# baseline (speedup 1.0000x reference)
import functools

import jax
import jax.numpy as jnp
from jax.experimental import pallas as pl
from jax.experimental.pallas import tpu as pltpu

BM = 400


def _layer1_body(adj_ref, xfull_ref, xi_ref, w0_ref, w1_ref, b_ref, h_ref):
    y = jnp.dot(adj_ref[...], xfull_ref[...], preferred_element_type=jnp.float32)
    h = (
        jnp.dot(xi_ref[...], w0_ref[...], preferred_element_type=jnp.float32)
        + jnp.dot(y, w1_ref[...], preferred_element_type=jnp.float32)
        + b_ref[...]
    )
    h_ref[...] = jnp.maximum(h, 0.0)


def _layer2_body(adj_ref, hfull_ref, hi_ref, w0_ref, w1_ref, b_ref, o_ref):
    z = jnp.dot(adj_ref[...], hfull_ref[...], preferred_element_type=jnp.float32)
    o = (
        jnp.dot(hi_ref[...], w0_ref[...], preferred_element_type=jnp.float32)
        + jnp.dot(z, w1_ref[...], preferred_element_type=jnp.float32)
        + b_ref[...]
    )
    m = jnp.max(o, axis=1, keepdims=True)
    e = jnp.exp(o - m)
    lse = jnp.log(jnp.sum(e, axis=1, keepdims=True))
    o_ref[...] = o - m - lse


def _cheb_layer(body, a_feat, adj, feat_dim_out, w0, w1, b):
    n, f_in = a_feat.shape
    grid = (n // BM,)
    return pl.pallas_call(
        body,
        grid=grid,
        in_specs=[
            pl.BlockSpec((BM, n), lambda i: (i, 0)),
            pl.BlockSpec((n, f_in), lambda i: (0, 0)),
            pl.BlockSpec((BM, f_in), lambda i: (i, 0)),
            pl.BlockSpec((f_in, feat_dim_out), lambda i: (0, 0)),
            pl.BlockSpec((f_in, feat_dim_out), lambda i: (0, 0)),
            pl.BlockSpec((1, feat_dim_out), lambda i: (0, 0)),
        ],
        out_specs=pl.BlockSpec((BM, feat_dim_out), lambda i: (i, 0)),
        out_shape=jax.ShapeDtypeStruct((n, feat_dim_out), jnp.float32),
    )(adj, a_feat, a_feat, w0, w1, b)


@jax.jit
def kernel(x, adj, W1_0, W1_1, b1, W2_0, W2_1, b2):
    hid = W1_0.shape[1]
    c_out = W2_0.shape[1]
    h = _cheb_layer(_layer1_body, x, adj, hid, W1_0, W1_1, b1.reshape(1, hid))
    return _cheb_layer(_layer2_body, h, adj, c_out, W2_0, W2_1, b2.reshape(1, c_out))
